# Initial kernel scaffold; baseline (speedup 1.0000x reference)
import jax
import jax.numpy as jnp
from jax.experimental import pallas as pl


def kernel(ctx_out, cand_emb, poly_code_weight):
    raise NotImplementedError("write your pallas kernel here")


# speedup vs baseline: 2.5009x; 2.5009x over previous
import jax
import jax.numpy as jnp
from jax.experimental import pallas as pl

B, S, R, D, M = 32, 512, 1024, 768, 64


def _poly_kernel(ctx_ref, cand_ref, w_ref, out_ref):
    ctx = ctx_ref[0]
    w = w_ref[...]
    logits = jnp.dot(w, ctx.T, preferred_element_type=jnp.float32)
    lmax = jnp.max(logits, axis=-1, keepdims=True)
    e = jnp.exp(logits - lmax)
    a = e / jnp.sum(e, axis=-1, keepdims=True)
    embs = jnp.dot(a, ctx, preferred_element_type=jnp.float32)
    cand = cand_ref[0]
    L = jnp.dot(cand, embs.T, preferred_element_type=jnp.float32)
    lm = jnp.max(L, axis=-1, keepdims=True)
    el = jnp.exp(L - lm)
    out = jnp.sum(el * L, axis=-1) / jnp.sum(el, axis=-1)
    out_ref[0, 0] = out


def kernel(ctx_out, cand_emb, poly_code_weight):
    out3 = pl.pallas_call(
        _poly_kernel,
        grid=(B,),
        in_specs=[
            pl.BlockSpec((1, S, D), lambda b: (b, 0, 0)),
            pl.BlockSpec((1, R, D), lambda b: (b, 0, 0)),
            pl.BlockSpec((M, D), lambda b: (0, 0)),
        ],
        out_specs=pl.BlockSpec((1, 1, R), lambda b: (b, 0, 0)),
        out_shape=jax.ShapeDtypeStruct((B, 1, R), jnp.float32),
    )(ctx_out, cand_emb, poly_code_weight)
    return out3.reshape(B, R)


# speedup vs baseline: 3.0817x; 1.2323x over previous
import jax
import jax.numpy as jnp
from jax.experimental import pallas as pl

B, S, R, D, M = 32, 512, 1024, 768, 64


def _poly_kernel(ctx_ref, cand_ref, w_ref, out_ref):
    ctx = ctx_ref[0]
    w = w_ref[...]
    logits = jax.lax.dot_general(w, ctx, (((1,), (1,)), ((), ())),
                                 preferred_element_type=jnp.float32)
    lmax = jnp.max(logits, axis=-1, keepdims=True)
    e = jnp.exp(logits - lmax)
    a = e / jnp.sum(e, axis=-1, keepdims=True)
    embs = jnp.dot(a, ctx, preferred_element_type=jnp.float32)
    cand = cand_ref[0]
    lt = jax.lax.dot_general(embs, cand, (((1,), (1,)), ((), ())),
                             preferred_element_type=jnp.float32)
    lm = jnp.max(lt, axis=0, keepdims=True)
    el = jnp.exp(lt - lm)
    out = jnp.sum(el * lt, axis=0) / jnp.sum(el, axis=0)
    out_ref[0, 0] = out


def kernel(ctx_out, cand_emb, poly_code_weight):
    out3 = pl.pallas_call(
        _poly_kernel,
        grid=(B,),
        in_specs=[
            pl.BlockSpec((1, S, D), lambda b: (b, 0, 0)),
            pl.BlockSpec((1, R, D), lambda b: (b, 0, 0)),
            pl.BlockSpec((M, D), lambda b: (0, 0)),
        ],
        out_specs=pl.BlockSpec((1, 1, R), lambda b: (b, 0, 0)),
        out_shape=jax.ShapeDtypeStruct((B, 1, R), jnp.float32),
    )(ctx_out, cand_emb, poly_code_weight)
    return out3.reshape(B, R)
